# baseline (device time: 780382 ns/iter reference)
import jax
import jax.numpy as jnp
from jax import lax
from jax.experimental import pallas as pl
from jax.experimental.pallas import tpu as pltpu

N_DEV = 16
M, K, N = 4096, 4096, 8192
CHUNK = M // N_DEV
HALF = N // 2
SUB = HALF // 2
N_STEPS = 2 * (N_DEV - 1)

_MESH = pl.DeviceIdType.MESH
_DOT_DIMS = (((1,), (0,)), ((), ()))


def kernel(x, w_mat, scale_x, scale_w):
    xa = x.astype(jnp.bfloat16)
    wa = w_mat.astype(jnp.bfloat16)

    def body(x_ref, w_ref, sx_ref, sw_ref, out_ref,
             comm_f, comm_b, stage_f, stage_b,
             send_f, recv_f, send_b, recv_b,
             store_f, store_b, credit_f, credit_b):
        me = lax.axis_index("i")
        left = lax.rem(me + N_DEV - 1, N_DEV)
        right = lax.rem(me + 1, N_DEV)
        scale = sx_ref[0] * sw_ref[0]

        def chunk_of(i):
            return lax.rem(i + 2 * N_DEV, N_DEV)

        def tile(idx, col0):
            return lax.dot_general(
                x_ref[pl.ds(idx * CHUNK, CHUNK), :],
                w_ref[:, pl.ds(col0, SUB)],
                _DOT_DIMS, preferred_element_type=jnp.float32)

        def store_cols(src, idx, col0, sem):
            cp = pltpu.make_async_copy(
                src, out_ref.at[pl.ds(idx * CHUNK, CHUNK), pl.ds(col0, SUB)],
                sem)
            cp.start()
            return cp

        barrier = pltpu.get_barrier_semaphore()
        for nbr in (left, right):
            pl.semaphore_signal(barrier, inc=1, device_id=(nbr,),
                                device_id_type=_MESH)
        pl.semaphore_wait(barrier, 2)

        comm_f[0, :, 0:SUB] = tile(me, 0).astype(jnp.bfloat16)
        comm_f[0, :, SUB:HALF] = tile(me, SUB).astype(jnp.bfloat16)
        comm_b[0, :, 0:SUB] = tile(me, HALF).astype(jnp.bfloat16)
        comm_b[0, :, SUB:HALF] = tile(me, HALF + SUB).astype(jnp.bfloat16)

        def signal_free(slot, sub):
            pl.semaphore_signal(credit_f.at[slot, sub], inc=1,
                                device_id=(left,), device_id_type=_MESH)
            pl.semaphore_signal(credit_b.at[slot, sub], inc=1,
                                device_id=(right,), device_id_type=_MESH)

        signal_free(1, 0)
        signal_free(1, 1)

        inflight = {}
        pending = {}

        for t in range(2 * N_STEPS + 1):
            if t < 2 * N_STEPS:
                sub = t % 2
                s = t // 2
                snd, rcv = s % 2, (s + 1) % 2
                off = sub * SUB
                pl.semaphore_wait(credit_f.at[rcv, sub], 1)
                pl.semaphore_wait(credit_b.at[rcv, sub], 1)
                rf = pltpu.make_async_remote_copy(
                    src_ref=comm_f.at[snd, :, pl.ds(off, SUB)],
                    dst_ref=comm_f.at[rcv, :, pl.ds(off, SUB)],
                    send_sem=send_f.at[snd, sub],
                    recv_sem=recv_f.at[rcv, sub],
                    device_id=(right,), device_id_type=_MESH)
                rb = pltpu.make_async_remote_copy(
                    src_ref=comm_b.at[snd, :, pl.ds(off, SUB)],
                    dst_ref=comm_b.at[rcv, :, pl.ds(off, SUB)],
                    send_sem=send_b.at[snd, sub],
                    recv_sem=recv_b.at[rcv, sub],
                    device_id=(left,), device_id_type=_MESH)
                rf.start()
                rb.start()
                pf = pb = None
                if s < N_DEV - 1:
                    pf = tile(chunk_of(me - s - 1), off)
                    pb = tile(chunk_of(me + s + 1), HALF + off)
                inflight[sub] = (rf, rb, pf, pb)

            if t >= 1:
                sub = (t - 1) % 2
                s = (t - 1) // 2
                snd, rcv = s % 2, (s + 1) % 2
                off = sub * SUB
                rf, rb, pf, pb = inflight[sub]
                rf.wait()
                rb.wait()
                if s + 1 < N_STEPS:
                    for cp in pending.pop((snd, sub), ()):
                        cp.wait()
                    signal_free(snd, sub)
                if s < N_DEV - 1:
                    acc_f = (comm_f[rcv, :, off:off + SUB]
                             .astype(jnp.float32) + pf)
                    acc_b = (comm_b[rcv, :, off:off + SUB]
                             .astype(jnp.float32) + pb)
                    if s == N_DEV - 2:
                        yf = jnp.maximum(acc_f * scale, 0.0).astype(
                            jnp.bfloat16)
                        yb = jnp.maximum(acc_b * scale, 0.0).astype(
                            jnp.bfloat16)
                        comm_f[rcv, :, off:off + SUB] = yf
                        comm_b[rcv, :, off:off + SUB] = yb
                        stage_f[sub] = yf.astype(jnp.float32)
                        stage_b[sub] = yb.astype(jnp.float32)
                        pending[(rcv, sub)] = [
                            store_cols(stage_f.at[sub], chunk_of(me + 1),
                                       off, store_f.at[rcv, sub]),
                            store_cols(stage_b.at[sub], chunk_of(me - 1),
                                       HALF + off, store_b.at[rcv, sub]),
                        ]
                    else:
                        comm_f[rcv, :, off:off + SUB] = acc_f.astype(
                            jnp.bfloat16)
                        comm_b[rcv, :, off:off + SUB] = acc_b.astype(
                            jnp.bfloat16)
                else:
                    tt = s - (N_DEV - 1)
                    stage_f[sub] = comm_f[rcv, :, off:off + SUB].astype(
                        jnp.float32)
                    stage_b[sub] = comm_b[rcv, :, off:off + SUB].astype(
                        jnp.float32)
                    pending[(rcv, sub)] = [
                        store_cols(stage_f.at[sub], chunk_of(me - tt), off,
                                   store_f.at[rcv, sub]),
                        store_cols(stage_b.at[sub], chunk_of(me + tt),
                                   HALF + off, store_b.at[rcv, sub]),
                    ]

        for cps in pending.values():
            for cp in cps:
                cp.wait()

    return pl.pallas_call(
        body,
        out_shape=jax.ShapeDtypeStruct((M, N), jnp.float32),
        in_specs=[
            pl.BlockSpec(memory_space=pltpu.VMEM),
            pl.BlockSpec(memory_space=pltpu.VMEM),
            pl.BlockSpec(memory_space=pltpu.SMEM),
            pl.BlockSpec(memory_space=pltpu.SMEM),
        ],
        out_specs=pl.BlockSpec(memory_space=pl.ANY),
        scratch_shapes=[
            pltpu.VMEM((2, CHUNK, HALF), jnp.bfloat16),
            pltpu.VMEM((2, CHUNK, HALF), jnp.bfloat16),
            pltpu.VMEM((2, CHUNK, SUB), jnp.float32),
            pltpu.VMEM((2, CHUNK, SUB), jnp.float32),
            pltpu.SemaphoreType.DMA((2, 2)),
            pltpu.SemaphoreType.DMA((2, 2)),
            pltpu.SemaphoreType.DMA((2, 2)),
            pltpu.SemaphoreType.DMA((2, 2)),
            pltpu.SemaphoreType.DMA((2, 2)),
            pltpu.SemaphoreType.DMA((2, 2)),
            pltpu.SemaphoreType.REGULAR((2, 2)),
            pltpu.SemaphoreType.REGULAR((2, 2)),
        ],
        compiler_params=pltpu.CompilerParams(collective_id=0),
    )(xa, wa, scale_x, scale_w)


# device time: 779677 ns/iter; 1.0009x vs baseline; 1.0009x over previous
import jax
import jax.numpy as jnp
from jax import lax
from jax.experimental import pallas as pl
from jax.experimental.pallas import tpu as pltpu

N_DEV = 16
M, K, N = 4096, 4096, 8192
CHUNK = M // N_DEV
HALF = N // 2
NLANES = 4
SUB = HALF // NLANES
N_STEPS = 2 * (N_DEV - 1)

_MESH = pl.DeviceIdType.MESH
_DOT_DIMS = (((1,), (0,)), ((), ()))


def kernel(x, w_mat, scale_x, scale_w):
    xa = x.astype(jnp.bfloat16)
    wa = w_mat.astype(jnp.bfloat16)

    def body(x_ref, w_ref, sx_ref, sw_ref, out_ref,
             comm_f, comm_b, stage_f, stage_b,
             send_f, recv_f, send_b, recv_b,
             store_f, store_b, credit_f, credit_b):
        me = lax.axis_index("i")
        left = lax.rem(me + N_DEV - 1, N_DEV)
        right = lax.rem(me + 1, N_DEV)
        scale = sx_ref[0] * sw_ref[0]

        def chunk_of(i):
            return lax.rem(i + 2 * N_DEV, N_DEV)

        def tile(idx, col0):
            return lax.dot_general(
                x_ref[pl.ds(idx * CHUNK, CHUNK), :],
                w_ref[:, pl.ds(col0, SUB)],
                _DOT_DIMS, preferred_element_type=jnp.float32)

        def store_cols(src, idx, col0, sem):
            cp = pltpu.make_async_copy(
                src, out_ref.at[pl.ds(idx * CHUNK, CHUNK), pl.ds(col0, SUB)],
                sem)
            cp.start()
            return cp

        barrier = pltpu.get_barrier_semaphore()
        for nbr in (left, right):
            pl.semaphore_signal(barrier, inc=1, device_id=(nbr,),
                                device_id_type=_MESH)
        pl.semaphore_wait(barrier, 2)

        for sub in range(NLANES):
            off = sub * SUB
            comm_f[0, :, off:off + SUB] = tile(me, off).astype(jnp.bfloat16)
            comm_b[0, :, off:off + SUB] = tile(me, HALF + off).astype(
                jnp.bfloat16)

        def signal_free(slot, sub):
            pl.semaphore_signal(credit_f.at[slot, sub], inc=1,
                                device_id=(left,), device_id_type=_MESH)
            pl.semaphore_signal(credit_b.at[slot, sub], inc=1,
                                device_id=(right,), device_id_type=_MESH)

        for sub in range(NLANES):
            signal_free(1, sub)

        inflight = {}
        pending = {}

        for t in range(NLANES * N_STEPS + 1):
            if t < NLANES * N_STEPS:
                sub = t % NLANES
                s = t // NLANES
                snd, rcv = s % 2, (s + 1) % 2
                off = sub * SUB
                pl.semaphore_wait(credit_f.at[rcv, sub], 1)
                pl.semaphore_wait(credit_b.at[rcv, sub], 1)
                rf = pltpu.make_async_remote_copy(
                    src_ref=comm_f.at[snd, :, pl.ds(off, SUB)],
                    dst_ref=comm_f.at[rcv, :, pl.ds(off, SUB)],
                    send_sem=send_f.at[snd, sub],
                    recv_sem=recv_f.at[rcv, sub],
                    device_id=(right,), device_id_type=_MESH)
                rb = pltpu.make_async_remote_copy(
                    src_ref=comm_b.at[snd, :, pl.ds(off, SUB)],
                    dst_ref=comm_b.at[rcv, :, pl.ds(off, SUB)],
                    send_sem=send_b.at[snd, sub],
                    recv_sem=recv_b.at[rcv, sub],
                    device_id=(left,), device_id_type=_MESH)
                rf.start()
                rb.start()
                pf = pb = None
                if s < N_DEV - 1:
                    pf = tile(chunk_of(me - s - 1), off)
                    pb = tile(chunk_of(me + s + 1), HALF + off)
                inflight[sub] = (rf, rb, pf, pb)

            if t >= 1:
                sub = (t - 1) % NLANES
                s = (t - 1) // NLANES
                snd, rcv = s % 2, (s + 1) % 2
                off = sub * SUB
                rf, rb, pf, pb = inflight[sub]
                rf.wait()
                rb.wait()
                if s + 1 < N_STEPS:
                    for cp in pending.pop((snd, sub), ()):
                        cp.wait()
                    signal_free(snd, sub)
                if s < N_DEV - 1:
                    acc_f = (comm_f[rcv, :, off:off + SUB]
                             .astype(jnp.float32) + pf)
                    acc_b = (comm_b[rcv, :, off:off + SUB]
                             .astype(jnp.float32) + pb)
                    if s == N_DEV - 2:
                        yf = jnp.maximum(acc_f * scale, 0.0).astype(
                            jnp.bfloat16)
                        yb = jnp.maximum(acc_b * scale, 0.0).astype(
                            jnp.bfloat16)
                        comm_f[rcv, :, off:off + SUB] = yf
                        comm_b[rcv, :, off:off + SUB] = yb
                        stage_f[sub] = yf.astype(jnp.float32)
                        stage_b[sub] = yb.astype(jnp.float32)
                        pending[(rcv, sub)] = [
                            store_cols(stage_f.at[sub], chunk_of(me + 1),
                                       off, store_f.at[rcv, sub]),
                            store_cols(stage_b.at[sub], chunk_of(me - 1),
                                       HALF + off, store_b.at[rcv, sub]),
                        ]
                    else:
                        comm_f[rcv, :, off:off + SUB] = acc_f.astype(
                            jnp.bfloat16)
                        comm_b[rcv, :, off:off + SUB] = acc_b.astype(
                            jnp.bfloat16)
                else:
                    tt = s - (N_DEV - 1)
                    stage_f[sub] = comm_f[rcv, :, off:off + SUB].astype(
                        jnp.float32)
                    stage_b[sub] = comm_b[rcv, :, off:off + SUB].astype(
                        jnp.float32)
                    pending[(rcv, sub)] = [
                        store_cols(stage_f.at[sub], chunk_of(me - tt), off,
                                   store_f.at[rcv, sub]),
                        store_cols(stage_b.at[sub], chunk_of(me + tt),
                                   HALF + off, store_b.at[rcv, sub]),
                    ]

        for cps in pending.values():
            for cp in cps:
                cp.wait()

    return pl.pallas_call(
        body,
        out_shape=jax.ShapeDtypeStruct((M, N), jnp.float32),
        in_specs=[
            pl.BlockSpec(memory_space=pltpu.VMEM),
            pl.BlockSpec(memory_space=pltpu.VMEM),
            pl.BlockSpec(memory_space=pltpu.SMEM),
            pl.BlockSpec(memory_space=pltpu.SMEM),
        ],
        out_specs=pl.BlockSpec(memory_space=pl.ANY),
        scratch_shapes=[
            pltpu.VMEM((2, CHUNK, HALF), jnp.bfloat16),
            pltpu.VMEM((2, CHUNK, HALF), jnp.bfloat16),
            pltpu.VMEM((NLANES, CHUNK, SUB), jnp.float32),
            pltpu.VMEM((NLANES, CHUNK, SUB), jnp.float32),
            pltpu.SemaphoreType.DMA((2, NLANES)),
            pltpu.SemaphoreType.DMA((2, NLANES)),
            pltpu.SemaphoreType.DMA((2, NLANES)),
            pltpu.SemaphoreType.DMA((2, NLANES)),
            pltpu.SemaphoreType.DMA((2, NLANES)),
            pltpu.SemaphoreType.DMA((2, NLANES)),
            pltpu.SemaphoreType.REGULAR((2, NLANES)),
            pltpu.SemaphoreType.REGULAR((2, NLANES)),
        ],
        compiler_params=pltpu.CompilerParams(collective_id=0),
    )(xa, wa, scale_x, scale_w)
